# baseline (speedup 1.0000x reference)
import functools

import jax
import jax.numpy as jnp
from jax import lax
from jax.experimental import pallas as pl
from jax.experimental.pallas import tpu as pltpu
from jax.experimental.pallas import tpu_sc as plsc

N = 10000
E = 320000
D = 128
H = 512

NC = 2
NS = 16
CHUNK = 128
NCHUNK = 2528
F0 = 142
F1 = 16
FMAX = max(F0, F1)
SPAN = 30
EPAD = NCHUNK * CHUNK
NPAD_E = EPAD - E
NACC = 10112
RPT = NACC // NS
ZR = 8

_mesh = plsc.VectorSubcoreMesh(core_axis_name="c", subcore_axis_name="s")


def _fill_zbuf(zbuf):
    z16 = jnp.zeros((16,), jnp.float32)
    for r in range(ZR):
        for c in range(8):
            zbuf[r, pl.ds(c * 16, 16)] = z16


def _zero_acc(zbuf, acc, row0, nrow, semz):
    nj = nrow // ZR

    def zfire(j, carry):
        pltpu.async_copy(zbuf, acc.at[pl.ds(row0 + j * ZR, ZR)], semz)
        return carry

    lax.fori_loop(0, nj, zfire, 0)

    def zdrain(j, carry):
        pltpu.make_async_copy(zbuf, acc.at[pl.ds(row0 + j * ZR, ZR)],
                              semz).wait()
        return carry

    lax.fori_loop(0, nj, zdrain, 0)


def _spans(n):
    return [(lo, min(SPAN, n - lo)) for lo in range(0, n, SPAN)]


def _load_span(srcix, dstix, sbuf, dbuf, base, lo, ln, semi):
    pltpu.async_copy(srcix.at[pl.ds(base + lo, ln)], sbuf.at[pl.ds(0, ln)],
                     semi)
    pltpu.async_copy(dstix.at[pl.ds(base + lo, ln)], dbuf.at[pl.ds(0, ln)],
                     semi)


def _wait_span(srcix, dstix, sbuf, dbuf, base, lo, ln, semi):
    pltpu.make_async_copy(srcix.at[pl.ds(base + lo, ln)],
                          sbuf.at[pl.ds(0, ln)], semi).wait()
    pltpu.make_async_copy(dstix.at[pl.ds(base + lo, ln)],
                          dbuf.at[pl.ds(0, ln)], semi).wait()


def _gs_core(table_hbm, sbuf, dbuf, rows0, rows1, sem0, sem1, acc, n):
    HC = CHUNK // 2

    def gath(j, rows, sem):
        pltpu.async_copy(table_hbm.at[sbuf.at[j].at[0, pl.ds(0, HC)]],
                         rows.at[pl.ds(0, HC)], sem)
        pltpu.async_copy(table_hbm.at[sbuf.at[j].at[0, pl.ds(HC, HC)]],
                         rows.at[pl.ds(HC, HC)], sem)

    def wait_gath(j, rows, sem):
        pltpu.make_async_copy(table_hbm.at[sbuf.at[j].at[0, pl.ds(0, HC)]],
                              rows.at[pl.ds(0, HC)], sem).wait()
        pltpu.make_async_copy(table_hbm.at[sbuf.at[j].at[0, pl.ds(HC, HC)]],
                              rows.at[pl.ds(HC, HC)], sem).wait()

    gath(0, rows0, sem0)
    if n > 1:
        gath(1, rows1, sem1)

    def pair(p, carry):
        j0 = p * 2
        for o, rows, sem in ((0, rows0, sem0), (1, rows1, sem1)):
            j = j0 + o
            wait_gath(j, rows, sem)
            pltpu.sync_copy(rows, acc.at[dbuf.at[j].at[0]], add=True)

            @pl.when(j + 2 < n)
            def _():
                gath(j + 2, rows, sem)
        return carry

    lax.fori_loop(0, n // 2, pair, 0)
    if n % 2:
        j = n - 1
        wait_gath(j, rows0, sem0)
        pltpu.sync_copy(rows0, acc.at[dbuf.at[j].at[0]], add=True)


def _agg_sweep(table_hbm, srcix, dstix, sA, sB, dA, dB, rows0, rows1,
               sem0, sem1, semi, acc, base, n):
    spans = _spans(n)
    _load_span(srcix, dstix, sA, dA, base, *spans[0], semi)
    _wait_span(srcix, dstix, sA, dA, base, *spans[0], semi)
    for si, (lo, ln) in enumerate(spans):
        sbuf, dbuf = (sA, dA) if si % 2 == 0 else (sB, dB)
        nxt = spans[si + 1] if si + 1 < len(spans) else None
        if nxt is not None:
            nsb, ndb = (sB, dB) if si % 2 == 0 else (sA, dA)
            _load_span(srcix, dstix, nsb, ndb, base, *nxt, semi)
        _gs_core(table_hbm, sbuf, dbuf, rows0, rows1, sem0, sem1, acc, ln)
        if nxt is not None:
            _wait_span(srcix, dstix, nsb, ndb, base, *nxt, semi)


def _deg_sweep(ones_rows, dstix, dA, dB, semi, semd, acc, base, n):
    spans = _spans(n)
    pltpu.async_copy(dstix.at[pl.ds(base, spans[0][1])],
                     dA.at[pl.ds(0, spans[0][1])], semi)
    pltpu.make_async_copy(dstix.at[pl.ds(base, spans[0][1])],
                          dA.at[pl.ds(0, spans[0][1])], semi).wait()
    for si, (lo, ln) in enumerate(spans):
        dbuf = dA if si % 2 == 0 else dB
        nxt = spans[si + 1] if si + 1 < len(spans) else None
        if nxt is not None:
            ndb = dB if si % 2 == 0 else dA
            pltpu.async_copy(dstix.at[pl.ds(base + nxt[0], nxt[1])],
                             ndb.at[pl.ds(0, nxt[1])], semi)

        def dfire(t, carry):
            pltpu.async_copy(ones_rows, acc.at[dbuf.at[t].at[0]], semd,
                             add=True)
            return carry

        lax.fori_loop(0, ln, dfire, 0)

        def ddrain(t, carry):
            pltpu.make_async_copy(ones_rows, acc.at[dbuf.at[t].at[0]],
                                  semd).wait()
            return carry

        lax.fori_loop(0, ln, ddrain, 0)
        if nxt is not None:
            pltpu.make_async_copy(dstix.at[pl.ds(base + nxt[0], nxt[1])],
                                  ndb.at[pl.ds(0, nxt[1])], semi).wait()


def _sc_pass1_body(src_hbm, dst_hbm, x_hbm, ones_hbm,
                   out1, outd, sA, sB, dA, dB, rows0, rows1, zbuf,
                   acc1, sem0, sem1, semi, semd):
    cid = lax.axis_index("c")
    sid = lax.axis_index("s")
    row0 = sid * RPT
    _fill_zbuf(zbuf)

    def core_work(ci, n, base):
        _zero_acc(zbuf, acc1, row0, RPT, semd)
        plsc.subcore_barrier()

        _agg_sweep(x_hbm, src_hbm, dst_hbm, sA, sB, dA, dB, rows0, rows1,
                   sem0, sem1, semi, acc1, base, n)
        plsc.subcore_barrier()
        pltpu.sync_copy(acc1.at[pl.ds(row0, RPT)],
                        out1.at[ci, pl.ds(row0, RPT)])

        _zero_acc(zbuf, acc1, row0, RPT, semd)
        pltpu.sync_copy(ones_hbm, rows0)
        plsc.subcore_barrier()
        _deg_sweep(rows0, dst_hbm, dA, dB, semi, semd, acc1, base, n)
        plsc.subcore_barrier()
        pltpu.sync_copy(acc1.at[pl.ds(row0, RPT)],
                        outd.at[ci, pl.ds(row0, RPT)])

    @pl.when(cid == 0)
    def _():
        core_work(0, F0, sid * F0)

    @pl.when(cid == 1)
    def _():
        core_work(1, F1, NS * F0 + sid * F1)


@functools.partial(
    pl.kernel,
    out_type=(
        jax.ShapeDtypeStruct((NC, NACC, 128), jnp.float32),
        jax.ShapeDtypeStruct((NC, NACC, 128), jnp.float32),
    ),
    mesh=_mesh,
    scratch_types=[
        pltpu.VMEM((SPAN, 1, CHUNK), jnp.int32),
        pltpu.VMEM((SPAN, 1, CHUNK), jnp.int32),
        pltpu.VMEM((SPAN, 1, CHUNK), jnp.int32),
        pltpu.VMEM((SPAN, 1, CHUNK), jnp.int32),
        pltpu.VMEM((CHUNK, 128), jnp.float32),
        pltpu.VMEM((CHUNK, 128), jnp.float32),
        pltpu.VMEM((ZR, 128), jnp.float32),
        pltpu.VMEM_SHARED((NACC, 128), jnp.float32),
        pltpu.SemaphoreType.DMA,
        pltpu.SemaphoreType.DMA,
        pltpu.SemaphoreType.DMA,
        pltpu.SemaphoreType.DMA,
    ],
)
def _sc_pass1(*args):
    _sc_pass1_body(*args)


def _sc_pass2_body(idx_hbm, dst_hbm, h_hbm,
                   out2, sA, sB, dA, dB, rows0, rows1, zbuf, acc,
                   sem0, sem1, semi, semd):
    cid = lax.axis_index("c")
    sid = lax.axis_index("s")
    row0 = sid * RPT
    _fill_zbuf(zbuf)

    def core_work(ci, n, base):
        for k in range(4):
            _zero_acc(zbuf, acc, row0, RPT, semd)
            plsc.subcore_barrier()
            _agg_sweep(h_hbm, idx_hbm.at[k], dst_hbm, sA, sB, dA, dB,
                       rows0, rows1, sem0, sem1, semi, acc, base, n)
            plsc.subcore_barrier()
            pltpu.sync_copy(acc.at[pl.ds(row0, RPT)],
                            out2.at[ci, k, pl.ds(row0, RPT)])

    @pl.when(cid == 0)
    def _():
        core_work(0, F0, sid * F0)

    @pl.when(cid == 1)
    def _():
        core_work(1, F1, NS * F0 + sid * F1)


@functools.partial(
    pl.kernel,
    out_type=jax.ShapeDtypeStruct((NC, 4, NACC, 128), jnp.float32),
    mesh=_mesh,
    scratch_types=[
        pltpu.VMEM((SPAN, 1, CHUNK), jnp.int32),
        pltpu.VMEM((SPAN, 1, CHUNK), jnp.int32),
        pltpu.VMEM((SPAN, 1, CHUNK), jnp.int32),
        pltpu.VMEM((SPAN, 1, CHUNK), jnp.int32),
        pltpu.VMEM((CHUNK, 128), jnp.float32),
        pltpu.VMEM((CHUNK, 128), jnp.float32),
        pltpu.VMEM((ZR, 128), jnp.float32),
        pltpu.VMEM_SHARED((NACC, 128), jnp.float32),
        pltpu.SemaphoreType.DMA,
        pltpu.SemaphoreType.DMA,
        pltpu.SemaphoreType.DMA,
        pltpu.SemaphoreType.DMA,
    ],
)
def _sc_pass2(*args):
    _sc_pass2_body(*args)


BR = 1000
GRID = N // BR


def _rdeg_block(d0, d1, pid):
    deg = d0[0] + d1[0]
    rows = lax.broadcasted_iota(jnp.int32, deg.shape, 0)
    corr = jnp.where((pid == 0) & (rows == 0), jnp.float32(NPAD_E), 0.0)
    deg = jnp.maximum(deg - corr, 1.0)
    return 1.0 / deg[:, 0:1]


def _row0_mask(pid):
    rows = lax.broadcasted_iota(jnp.int32, (BR, 1), 0)
    return jnp.where((pid == 0) & (rows == 0), jnp.float32(NPAD_E), 0.0)


def _dot(a, b):
    return jnp.dot(a.astype(jnp.bfloat16), b.astype(jnp.bfloat16),
                   preferred_element_type=jnp.float32)


def _tc1_body(p0, p1, d0, d1, x0, w1, b1, h_out, ht_out):
    pid = pl.program_id(0)
    rdeg = _rdeg_block(d0, d1, pid)
    p = p0[0] + p1[0] - _row0_mask(pid) * x0[...]
    acc = _dot(p, w1[...])
    h = jnp.maximum(acc * rdeg + b1[...], 0.0)
    h_out[...] = h
    ht_out[...] = h.reshape(4 * BR, 128)


def _tc1(out1, outd, x0, W1, b1r):
    return pl.pallas_call(
        _tc1_body,
        grid=(GRID,),
        in_specs=[
            pl.BlockSpec((1, BR, 128), lambda i: (0, i, 0)),
            pl.BlockSpec((1, BR, 128), lambda i: (1, i, 0)),
            pl.BlockSpec((1, BR, 128), lambda i: (0, i, 0)),
            pl.BlockSpec((1, BR, 128), lambda i: (1, i, 0)),
            pl.BlockSpec((1, 128), lambda i: (0, 0)),
            pl.BlockSpec((128, H), lambda i: (0, 0)),
            pl.BlockSpec((1, H), lambda i: (0, 0)),
        ],
        out_specs=[
            pl.BlockSpec((BR, H), lambda i: (i, 0)),
            pl.BlockSpec((4 * BR, 128), lambda i: (i, 0)),
        ],
        out_shape=[
            jax.ShapeDtypeStruct((N, H), jnp.float32),
            jax.ShapeDtypeStruct((4 * N, 128), jnp.float32),
        ],
    )(out1, out1, outd, outd, x0, W1, b1r)


def _tc2_body(p8, h, d0, d1, w2, b2, wa1, ba1, wa2, ba2, wa3, ba3, wa4, ba4,
              out):
    pid = pl.program_id(0)
    rdeg = _rdeg_block(d0, d1, pid)
    m0 = _row0_mask(pid)
    acc = jnp.zeros((BR, H), jnp.float32)
    for k in range(4):
        pk = p8[k] + p8[4 + k] - m0 * h[0:1, pl.ds(k * 128, 128)]
        acc += _dot(pk, w2[pl.ds(k * 128, 128), :])
    h2 = h[...] + jnp.maximum(acc * rdeg + b2[...], 0.0)
    a = jnp.maximum(_dot(h2, wa1[...]) + ba1[...], 0.0)
    a = jnp.maximum(_dot(a, wa2[...]) + ba2[...], 0.0)
    a = jnp.maximum(_dot(a, wa3[...]) + ba3[...], 0.0)
    out[...] = _dot(a, wa4[...]) + ba4[...]


def _tc2(p8, h, outd, W2, b2r, Wa1, ba1r, Wa2, ba2r, Wa3, ba3r, Wa4p, ba4r):
    const = lambda shape: pl.BlockSpec(shape, lambda i: tuple(0 for _ in shape))
    return pl.pallas_call(
        _tc2_body,
        grid=(GRID,),
        in_specs=[
            pl.BlockSpec((8, BR, 128), lambda i: (0, i, 0)),
            pl.BlockSpec((BR, H), lambda i: (i, 0)),
            pl.BlockSpec((1, BR, 128), lambda i: (0, i, 0)),
            pl.BlockSpec((1, BR, 128), lambda i: (1, i, 0)),
            const((H, H)),
            const((1, H)),
            const((H, 1024)),
            const((1, 1024)),
            const((1024, H)),
            const((1, H)),
            const((H, 256)),
            const((1, 256)),
            const((256, 128)),
            const((1, 128)),
        ],
        out_specs=pl.BlockSpec((BR, 128), lambda i: (i, 0)),
        out_shape=jax.ShapeDtypeStruct((N, 128), jnp.float32),
    )(p8, h, outd, outd, W2, b2r, Wa1, ba1r, Wa2, ba2r, Wa3, ba3r, Wa4p,
      ba4r)


def kernel(nodes, edges, W1, b1, W2, b2, Wa1, ba1, Wa2, ba2, Wa3, ba3, Wa4,
           ba4):
    src = edges[0]
    dst = edges[1]
    src_p = jnp.concatenate(
        [src, jnp.zeros((NPAD_E,), jnp.int32)]).reshape(NCHUNK, 1, CHUNK)
    dst_p = jnp.concatenate(
        [dst, jnp.zeros((NPAD_E,), jnp.int32)]).reshape(NCHUNK, 1, CHUNK)
    idx2 = (src_p[None] * 4
            + jnp.arange(4, dtype=jnp.int32)[:, None, None, None])
    ones_c = jnp.ones((CHUNK, 128), jnp.float32)

    out1, outd = _sc_pass1(src_p, dst_p, nodes, ones_c)
    h, ht = _tc1(out1, outd, nodes[0:1], W1, b1.reshape(1, H))

    out2 = _sc_pass2(idx2, dst_p, ht)

    Wa4p = jnp.pad(Wa4, ((0, 0), (0, 123)))
    ba4r = jnp.pad(ba4, (0, 123)).reshape(1, 128)
    outp = _tc2(out2.reshape(8, NACC, 128), h, outd, W2, b2.reshape(1, H),
                Wa1, ba1.reshape(1, 1024), Wa2, ba2.reshape(1, H),
                Wa3, ba3.reshape(1, 256), Wa4p, ba4r)
    return outp[:, :5]


# speedup vs baseline: 1.1538x; 1.1538x over previous
import functools

import jax
import jax.numpy as jnp
from jax import lax
from jax.experimental import pallas as pl
from jax.experimental.pallas import tpu as pltpu
from jax.experimental.pallas import tpu_sc as plsc

N = 10000
E = 320000
D = 128
H = 512

NC = 2
NS = 16
CHUNK = 96
NCHUNK = 3360
F0 = 189
F1 = 21
SPAN = 23
EPAD = NCHUNK * CHUNK
NPAD_E = EPAD - E
NACC = 10112
RPT = NACC // NS
ZR = 8

_mesh = plsc.VectorSubcoreMesh(core_axis_name="c", subcore_axis_name="s")


def _fill_zbuf(zbuf):
    z16 = jnp.zeros((16,), jnp.float32)
    for r in range(ZR):
        for c in range(8):
            zbuf[r, pl.ds(c * 16, 16)] = z16


def _zero_acc(zbuf, acc, row0, nrow, semz):
    nj = nrow // ZR

    def zfire(j, carry):
        pltpu.async_copy(zbuf, acc.at[pl.ds(row0 + j * ZR, ZR)], semz)
        return carry

    lax.fori_loop(0, nj, zfire, 0)

    def zdrain(j, carry):
        pltpu.make_async_copy(zbuf, acc.at[pl.ds(row0 + j * ZR, ZR)],
                              semz).wait()
        return carry

    lax.fori_loop(0, nj, zdrain, 0)


def _spans(n):
    return [(lo, min(SPAN, n - lo)) for lo in range(0, n, SPAN)]


def _load_span(srcix, dstix, sbuf, dbuf, base, lo, ln, semi):
    pltpu.async_copy(srcix.at[pl.ds(base + lo, ln)], sbuf.at[pl.ds(0, ln)],
                     semi)
    pltpu.async_copy(dstix.at[pl.ds(base + lo, ln)], dbuf.at[pl.ds(0, ln)],
                     semi)


def _wait_span(srcix, dstix, sbuf, dbuf, base, lo, ln, semi):
    pltpu.make_async_copy(srcix.at[pl.ds(base + lo, ln)],
                          sbuf.at[pl.ds(0, ln)], semi).wait()
    pltpu.make_async_copy(dstix.at[pl.ds(base + lo, ln)],
                          dbuf.at[pl.ds(0, ln)], semi).wait()


def _gs_core(table_hbm, sbuf, dbuf, rows, sems, acc, n):
    def gath(j, slot):
        pltpu.async_copy(table_hbm.at[sbuf.at[j].at[0]], rows[slot],
                         sems[slot])

    def wait_gath(j, slot):
        pltpu.make_async_copy(table_hbm.at[sbuf.at[j].at[0]], rows[slot],
                              sems[slot]).wait()

    def scat(j, slot):
        pltpu.sync_copy(rows[slot], acc.at[dbuf.at[j].at[0]], add=True)

    NB = len(rows)
    for j in range(min(NB, n)):
        gath(j, j)

    def trip(p, carry):
        j0 = p * NB
        for o in range(NB):
            j = j0 + o
            wait_gath(j, o)
            scat(j, o)

            @pl.when(j + NB < n)
            def _():
                gath(j + NB, o)
        return carry

    lax.fori_loop(0, n // NB, trip, 0)
    for o in range(n % NB):
        j = (n // NB) * NB + o
        wait_gath(j, o)
        scat(j, o)


def _agg_sweep(table_hbm, srcix, dstix, sA, sB, dA, dB, rows, sems,
               semi, acc, base, n):
    spans = _spans(n)
    _load_span(srcix, dstix, sA, dA, base, *spans[0], semi)
    _wait_span(srcix, dstix, sA, dA, base, *spans[0], semi)
    for si, (lo, ln) in enumerate(spans):
        sbuf, dbuf = (sA, dA) if si % 2 == 0 else (sB, dB)
        nxt = spans[si + 1] if si + 1 < len(spans) else None
        if nxt is not None:
            nsb, ndb = (sB, dB) if si % 2 == 0 else (sA, dA)
            _load_span(srcix, dstix, nsb, ndb, base, *nxt, semi)
        _gs_core(table_hbm, sbuf, dbuf, rows, sems, acc, ln)
        if nxt is not None:
            _wait_span(srcix, dstix, nsb, ndb, base, *nxt, semi)


def _deg_sweep(ones_rows, dstix, dA, dB, semi, semd, acc, base, n):
    spans = _spans(n)
    pltpu.async_copy(dstix.at[pl.ds(base, spans[0][1])],
                     dA.at[pl.ds(0, spans[0][1])], semi)
    pltpu.make_async_copy(dstix.at[pl.ds(base, spans[0][1])],
                          dA.at[pl.ds(0, spans[0][1])], semi).wait()
    for si, (lo, ln) in enumerate(spans):
        dbuf = dA if si % 2 == 0 else dB
        nxt = spans[si + 1] if si + 1 < len(spans) else None
        if nxt is not None:
            ndb = dB if si % 2 == 0 else dA
            pltpu.async_copy(dstix.at[pl.ds(base + nxt[0], nxt[1])],
                             ndb.at[pl.ds(0, nxt[1])], semi)

        def dfire(t, carry):
            pltpu.async_copy(ones_rows, acc.at[dbuf.at[t].at[0]], semd,
                             add=True)
            return carry

        lax.fori_loop(0, ln, dfire, 0)

        def ddrain(t, carry):
            pltpu.make_async_copy(ones_rows, acc.at[dbuf.at[t].at[0]],
                                  semd).wait()
            return carry

        lax.fori_loop(0, ln, ddrain, 0)
        if nxt is not None:
            pltpu.make_async_copy(dstix.at[pl.ds(base + nxt[0], nxt[1])],
                                  ndb.at[pl.ds(0, nxt[1])], semi).wait()


def _sc_pass1_body(src_hbm, dst_hbm, x_hbm, ones_hbm,
                   out1, outd, sA, sB, dA, dB, rows0, rows1, rows2, zbuf,
                   acc1, sem0, sem1, sem2, semi, semd):
    rows = (rows0, rows1, rows2)
    sems = (sem0, sem1, sem2)
    cid = lax.axis_index("c")
    sid = lax.axis_index("s")
    row0 = sid * RPT
    _fill_zbuf(zbuf)

    def core_work(ci, n, base):
        _zero_acc(zbuf, acc1, row0, RPT, semd)
        plsc.subcore_barrier()

        _agg_sweep(x_hbm, src_hbm, dst_hbm, sA, sB, dA, dB, rows, sems,
                   semi, acc1, base, n)
        plsc.subcore_barrier()
        pltpu.sync_copy(acc1.at[pl.ds(row0, RPT)],
                        out1.at[ci, pl.ds(row0, RPT)])

        _zero_acc(zbuf, acc1, row0, RPT, semd)
        pltpu.sync_copy(ones_hbm, rows0)
        plsc.subcore_barrier()
        _deg_sweep(rows0, dst_hbm, dA, dB, semi, semd, acc1, base, n)
        plsc.subcore_barrier()
        pltpu.sync_copy(acc1.at[pl.ds(row0, RPT)],
                        outd.at[ci, pl.ds(row0, RPT)])

    @pl.when(cid == 0)
    def _():
        core_work(0, F0, sid * F0)

    @pl.when(cid == 1)
    def _():
        core_work(1, F1, NS * F0 + sid * F1)


@functools.partial(
    pl.kernel,
    out_type=(
        jax.ShapeDtypeStruct((NC, NACC, 128), jnp.float32),
        jax.ShapeDtypeStruct((NC, NACC, 128), jnp.float32),
    ),
    mesh=_mesh,
    scratch_types=[
        pltpu.VMEM((SPAN, 1, CHUNK), jnp.int32),
        pltpu.VMEM((SPAN, 1, CHUNK), jnp.int32),
        pltpu.VMEM((SPAN, 1, CHUNK), jnp.int32),
        pltpu.VMEM((SPAN, 1, CHUNK), jnp.int32),
        pltpu.VMEM((CHUNK, 128), jnp.float32),
        pltpu.VMEM((CHUNK, 128), jnp.float32),
        pltpu.VMEM((CHUNK, 128), jnp.float32),
        pltpu.VMEM((ZR, 128), jnp.float32),
        pltpu.VMEM_SHARED((NACC, 128), jnp.float32),
        pltpu.SemaphoreType.DMA,
        pltpu.SemaphoreType.DMA,
        pltpu.SemaphoreType.DMA,
        pltpu.SemaphoreType.DMA,
        pltpu.SemaphoreType.DMA,
    ],
)
def _sc_pass1(*args):
    _sc_pass1_body(*args)


def _sc_pass2_body(idx_hbm, dst_hbm, h_hbm,
                   out2, sA, sB, dA, dB, rows0, rows1, rows2, zbuf, acc,
                   sem0, sem1, sem2, semi, semd):
    rows = (rows0, rows1, rows2)
    sems = (sem0, sem1, sem2)
    cid = lax.axis_index("c")
    sid = lax.axis_index("s")
    row0 = sid * RPT
    _fill_zbuf(zbuf)

    def core_work(ci, n, base):
        for k in range(4):
            _zero_acc(zbuf, acc, row0, RPT, semd)
            plsc.subcore_barrier()
            _agg_sweep(h_hbm, idx_hbm.at[k], dst_hbm, sA, sB, dA, dB,
                       rows, sems, semi, acc, base, n)
            plsc.subcore_barrier()
            pltpu.sync_copy(acc.at[pl.ds(row0, RPT)],
                            out2.at[ci, k, pl.ds(row0, RPT)])

    @pl.when(cid == 0)
    def _():
        core_work(0, F0, sid * F0)

    @pl.when(cid == 1)
    def _():
        core_work(1, F1, NS * F0 + sid * F1)


@functools.partial(
    pl.kernel,
    out_type=jax.ShapeDtypeStruct((NC, 4, NACC, 128), jnp.float32),
    mesh=_mesh,
    scratch_types=[
        pltpu.VMEM((SPAN, 1, CHUNK), jnp.int32),
        pltpu.VMEM((SPAN, 1, CHUNK), jnp.int32),
        pltpu.VMEM((SPAN, 1, CHUNK), jnp.int32),
        pltpu.VMEM((SPAN, 1, CHUNK), jnp.int32),
        pltpu.VMEM((CHUNK, 128), jnp.float32),
        pltpu.VMEM((CHUNK, 128), jnp.float32),
        pltpu.VMEM((CHUNK, 128), jnp.float32),
        pltpu.VMEM((ZR, 128), jnp.float32),
        pltpu.VMEM_SHARED((NACC, 128), jnp.float32),
        pltpu.SemaphoreType.DMA,
        pltpu.SemaphoreType.DMA,
        pltpu.SemaphoreType.DMA,
        pltpu.SemaphoreType.DMA,
        pltpu.SemaphoreType.DMA,
    ],
)
def _sc_pass2(*args):
    _sc_pass2_body(*args)


BR = 1000
GRID = N // BR


def _rdeg_block(d0, d1, pid):
    deg = d0[0] + d1[0]
    rows = lax.broadcasted_iota(jnp.int32, deg.shape, 0)
    corr = jnp.where((pid == 0) & (rows == 0), jnp.float32(NPAD_E), 0.0)
    deg = jnp.maximum(deg - corr, 1.0)
    return 1.0 / deg[:, 0:1]


def _row0_mask(pid):
    rows = lax.broadcasted_iota(jnp.int32, (BR, 1), 0)
    return jnp.where((pid == 0) & (rows == 0), jnp.float32(NPAD_E), 0.0)


def _dot(a, b):
    return jnp.dot(a.astype(jnp.bfloat16), b.astype(jnp.bfloat16),
                   preferred_element_type=jnp.float32)


def _tc1_body(p0, p1, d0, d1, x0, w1, b1, h_out, ht_out):
    pid = pl.program_id(0)
    rdeg = _rdeg_block(d0, d1, pid)
    p = p0[0] + p1[0] - _row0_mask(pid) * x0[...]
    acc = _dot(p, w1[...])
    h = jnp.maximum(acc * rdeg + b1[...], 0.0)
    h_out[...] = h
    ht_out[...] = h.reshape(4 * BR, 128)


def _tc1(out1, outd, x0, W1, b1r):
    return pl.pallas_call(
        _tc1_body,
        grid=(GRID,),
        in_specs=[
            pl.BlockSpec((1, BR, 128), lambda i: (0, i, 0)),
            pl.BlockSpec((1, BR, 128), lambda i: (1, i, 0)),
            pl.BlockSpec((1, BR, 128), lambda i: (0, i, 0)),
            pl.BlockSpec((1, BR, 128), lambda i: (1, i, 0)),
            pl.BlockSpec((1, 128), lambda i: (0, 0)),
            pl.BlockSpec((128, H), lambda i: (0, 0)),
            pl.BlockSpec((1, H), lambda i: (0, 0)),
        ],
        out_specs=[
            pl.BlockSpec((BR, H), lambda i: (i, 0)),
            pl.BlockSpec((4 * BR, 128), lambda i: (i, 0)),
        ],
        out_shape=[
            jax.ShapeDtypeStruct((N, H), jnp.float32),
            jax.ShapeDtypeStruct((4 * N, 128), jnp.float32),
        ],
    )(out1, out1, outd, outd, x0, W1, b1r)


def _tc2_body(p8, h, d0, d1, w2, b2, wa1, ba1, wa2, ba2, wa3, ba3, wa4, ba4,
              out):
    pid = pl.program_id(0)
    rdeg = _rdeg_block(d0, d1, pid)
    m0 = _row0_mask(pid)
    acc = jnp.zeros((BR, H), jnp.float32)
    for k in range(4):
        pk = p8[k] + p8[4 + k] - m0 * h[0:1, pl.ds(k * 128, 128)]
        acc += _dot(pk, w2[pl.ds(k * 128, 128), :])
    h2 = h[...] + jnp.maximum(acc * rdeg + b2[...], 0.0)
    a = jnp.maximum(_dot(h2, wa1[...]) + ba1[...], 0.0)
    a = jnp.maximum(_dot(a, wa2[...]) + ba2[...], 0.0)
    a = jnp.maximum(_dot(a, wa3[...]) + ba3[...], 0.0)
    out[...] = _dot(a, wa4[...]) + ba4[...]


def _tc2(p8, h, outd, W2, b2r, Wa1, ba1r, Wa2, ba2r, Wa3, ba3r, Wa4p, ba4r):
    const = lambda shape: pl.BlockSpec(shape, lambda i: tuple(0 for _ in shape))
    return pl.pallas_call(
        _tc2_body,
        grid=(GRID,),
        in_specs=[
            pl.BlockSpec((8, BR, 128), lambda i: (0, i, 0)),
            pl.BlockSpec((BR, H), lambda i: (i, 0)),
            pl.BlockSpec((1, BR, 128), lambda i: (0, i, 0)),
            pl.BlockSpec((1, BR, 128), lambda i: (1, i, 0)),
            const((H, H)),
            const((1, H)),
            const((H, 1024)),
            const((1, 1024)),
            const((1024, H)),
            const((1, H)),
            const((H, 256)),
            const((1, 256)),
            const((256, 128)),
            const((1, 128)),
        ],
        out_specs=pl.BlockSpec((BR, 128), lambda i: (i, 0)),
        out_shape=jax.ShapeDtypeStruct((N, 128), jnp.float32),
    )(p8, h, outd, outd, W2, b2r, Wa1, ba1r, Wa2, ba2r, Wa3, ba3r, Wa4p,
      ba4r)


def kernel(nodes, edges, W1, b1, W2, b2, Wa1, ba1, Wa2, ba2, Wa3, ba3, Wa4,
           ba4):
    src = edges[0]
    dst = edges[1]
    src_p = jnp.concatenate(
        [src, jnp.zeros((NPAD_E,), jnp.int32)]).reshape(NCHUNK, 1, CHUNK)
    dst_p = jnp.concatenate(
        [dst, jnp.zeros((NPAD_E,), jnp.int32)]).reshape(NCHUNK, 1, CHUNK)
    idx2 = (src_p[None] * 4
            + jnp.arange(4, dtype=jnp.int32)[:, None, None, None])
    ones_c = jnp.ones((CHUNK, 128), jnp.float32)

    out1, outd = _sc_pass1(src_p, dst_p, nodes, ones_c)
    h, ht = _tc1(out1, outd, nodes[0:1], W1, b1.reshape(1, H))

    out2 = _sc_pass2(idx2, dst_p, ht)

    Wa4p = jnp.pad(Wa4, ((0, 0), (0, 123)))
    ba4r = jnp.pad(ba4, (0, 123)).reshape(1, 128)
    outp = _tc2(out2.reshape(8, NACC, 128), h, outd, W2, b2.reshape(1, H),
                Wa1, ba1.reshape(1, 1024), Wa2, ba2.reshape(1, H),
                Wa3, ba3.reshape(1, 256), Wa4p, ba4r)
    return outp[:, :5]
